# baseline (speedup 1.0000x reference)
import functools

import jax
import jax.numpy as jnp
from jax import lax
from jax.experimental import pallas as pl
from jax.experimental.pallas import tpu as pltpu
from jax.experimental.pallas import tpu_sc as plsc

NC = 2
NS = 16
NW = NC * NS
LANES = 16
CHUNK = 80


@functools.lru_cache(maxsize=None)
def _make_stage1(E, N, H):
    EP = E // NW
    nch = EP // CHUNK
    RPT = (N // NS) & ~7
    tail = N - RPT * NS
    zrows = 48
    assert EP % CHUNK == 0 and RPT % zrows == 0 and tail % 8 == 0
    assert tail <= zrows

    mesh = plsc.VectorSubcoreMesh(core_axis_name="c", subcore_axis_name="s")

    def body(e_hbm, start_hbm, end_hbm, vol_hbm,
             msgp_hbm, combo_hbm, idx1_hbm, idx2_hbm,
             vol_v, erows, sidx, eidx, combo_v, i1_v, i2_v, zblk, msg_sh,
             lsem, ssem, wsem):
        cid = lax.axis_index("c")
        sid = lax.axis_index("s")
        wid = cid * NS + sid

        zero16 = jnp.zeros((LANES,), jnp.float32)
        for i in range(zrows):
            for j in range(H // LANES):
                zblk[i, pl.ds(j * LANES, LANES)] = zero16
        r0 = pl.multiple_of(sid * RPT, 8)
        for k in range(RPT // zrows):
            pltpu.sync_copy(zblk, msg_sh.at[pl.ds(r0 + k * zrows, zrows)])
        if tail:
            @pl.when(sid == 0)
            def _():
                pltpu.sync_copy(zblk.at[pl.ds(0, tail)],
                                msg_sh.at[pl.ds(RPT * NS, tail)])

        pltpu.sync_copy(vol_hbm, vol_v)
        plsc.subcore_barrier()

        def chunk_base(t):
            return pl.multiple_of(wid * EP + t * CHUNK, 8)

        def start_loads(t, b):
            base = chunk_base(t)
            pltpu.async_copy(start_hbm.at[pl.ds(base, CHUNK)], sidx.at[b], lsem)
            pltpu.async_copy(end_hbm.at[pl.ds(base, CHUNK)], eidx.at[b], lsem)
            pltpu.async_copy(e_hbm.at[pl.ds(base, CHUNK)], erows.at[b], lsem)

        def wait_loads(t, b):
            base = chunk_base(t)
            pltpu.make_async_copy(start_hbm.at[pl.ds(base, CHUNK)],
                                  sidx.at[b], lsem).wait()
            pltpu.make_async_copy(end_hbm.at[pl.ds(base, CHUNK)],
                                  eidx.at[b], lsem).wait()
            pltpu.make_async_copy(e_hbm.at[pl.ds(base, CHUNK)],
                                  erows.at[b], lsem).wait()

        def wait_scatter(b):
            pltpu.make_async_copy(erows.at[b], msg_sh.at[eidx.at[b]],
                                  ssem).wait()

        def wait_writes(t, b):
            base = chunk_base(t)
            pltpu.make_async_copy(combo_v.at[b],
                                  combo_hbm.at[pl.ds(base, CHUNK)], wsem).wait()
            pltpu.make_async_copy(i1_v.at[b],
                                  idx1_hbm.at[pl.ds(base, CHUNK)], wsem).wait()
            pltpu.make_async_copy(i2_v.at[b],
                                  idx2_hbm.at[pl.ds(base, CHUNK)], wsem).wait()

        start_loads(0, 0)

        @pl.loop(0, (nch + 1) // 2)
        def _(p):
            for b in range(2):
                t = p * 2 + b

                @pl.when(t < nch)
                def _():
                    base = chunk_base(t)
                    wait_loads(t, b)
                    pltpu.async_copy(erows.at[b], msg_sh.at[eidx.at[b]], ssem,
                                     add=True)

                    @pl.when(t >= 1)
                    def _():
                        wait_scatter(1 - b)

                    @pl.when(t + 1 < nch)
                    def _():
                        start_loads(t + 1, 1 - b)

                    @pl.when(t >= 2)
                    def _():
                        wait_writes(t - 2, b)

                    for j in range(CHUNK // LANES):
                        sl = (b, pl.ds(j * LANES, LANES))
                        s = sidx[sl]
                        d = eidx[sl]
                        vs = plsc.load_gather(vol_v, [s])
                        ve = plsc.load_gather(vol_v, [d])
                        code = vs * 2 + ve
                        c = jnp.where(code == 3, 2,
                                      jnp.where(code == 2, 3, code))
                        cc = jnp.where(c == 3, 0, c)
                        combo_v[sl] = c
                        i1_v[sl] = cc * N + s
                        i2_v[sl] = cc * N + d
                    pltpu.async_copy(combo_v.at[b],
                                     combo_hbm.at[pl.ds(base, CHUNK)], wsem)
                    pltpu.async_copy(i1_v.at[b],
                                     idx1_hbm.at[pl.ds(base, CHUNK)], wsem)
                    pltpu.async_copy(i2_v.at[b],
                                     idx2_hbm.at[pl.ds(base, CHUNK)], wsem)

        wait_scatter((nch - 1) % 2)
        for t in (nch - 2, nch - 1):
            wait_writes(t, t % 2)
        plsc.subcore_barrier()
        pltpu.sync_copy(msg_sh.at[pl.ds(r0, RPT)],
                        msgp_hbm.at[cid, pl.ds(r0, RPT)])
        if tail:
            @pl.when(sid == 0)
            def _():
                pltpu.sync_copy(msg_sh.at[pl.ds(RPT * NS, tail)],
                                msgp_hbm.at[cid, pl.ds(RPT * NS, tail)])

    return pl.kernel(
        body,
        compiler_params=pltpu.CompilerParams(needs_layout_passes=False),
        out_type=[
            jax.ShapeDtypeStruct((NC, N, H), jnp.float32),
            jax.ShapeDtypeStruct((E,), jnp.int32),
            jax.ShapeDtypeStruct((E,), jnp.int32),
            jax.ShapeDtypeStruct((E,), jnp.int32),
        ],
        mesh=mesh,
        scratch_types=[
            pltpu.VMEM((N,), jnp.int32),
            pltpu.VMEM((2, CHUNK, H), jnp.float32),
            pltpu.VMEM((2, CHUNK), jnp.int32),
            pltpu.VMEM((2, CHUNK), jnp.int32),
            pltpu.VMEM((2, CHUNK), jnp.int32),
            pltpu.VMEM((2, CHUNK), jnp.int32),
            pltpu.VMEM((2, CHUNK), jnp.int32),
            pltpu.VMEM((48, H), jnp.float32),
            pltpu.VMEM_SHARED((N, H), jnp.float32),
            pltpu.SemaphoreType.DMA,
            pltpu.SemaphoreType.DMA,
            pltpu.SemaphoreType.DMA,
        ],
    )


@functools.lru_cache(maxsize=None)
def _make_stage3(E, N, H):
    EP = E // NW
    nch = EP // CHUNK
    HW = H // 2

    mesh = plsc.VectorSubcoreMesh(core_axis_name="c", subcore_axis_name="s")

    def body(pa_hbm, qb_hbm, idx1_hbm, idx2_hbm, g_hbm,
             i1_v, i2_v, r1, r2, go, gsem, isem, osem):
        cid = lax.axis_index("c")
        sid = lax.axis_index("s")
        wid = cid * NS + sid

        def chunk_base(t):
            return pl.multiple_of(wid * EP + t * CHUNK, 8)

        def load_idx(t, b):
            base = chunk_base(t)
            pltpu.async_copy(idx1_hbm.at[pl.ds(base, CHUNK)], i1_v.at[b], isem)
            pltpu.async_copy(idx2_hbm.at[pl.ds(base, CHUNK)], i2_v.at[b], isem)

        def start_gather(b):
            pltpu.async_copy(pa_hbm.at[i1_v.at[b]], r1.at[b], gsem)
            pltpu.async_copy(qb_hbm.at[i2_v.at[b]], r2.at[b], gsem)

        def wait_idx(t, b):
            base = chunk_base(t)
            pltpu.make_async_copy(idx1_hbm.at[pl.ds(base, CHUNK)],
                                  i1_v.at[b], isem).wait()
            pltpu.make_async_copy(idx2_hbm.at[pl.ds(base, CHUNK)],
                                  i2_v.at[b], isem).wait()

        def wait_gather(b):
            pltpu.make_async_copy(pa_hbm.at[i1_v.at[b]], r1.at[b], gsem).wait()
            pltpu.make_async_copy(qb_hbm.at[i2_v.at[b]], r2.at[b], gsem).wait()

        def wait_write(t, b):
            pltpu.make_async_copy(go.at[b],
                                  g_hbm.at[pl.ds(chunk_base(t), CHUNK)],
                                  osem).wait()

        load_idx(0, 0)
        wait_idx(0, 0)
        start_gather(0)
        load_idx(1, 1)

        @pl.loop(0, (nch + 1) // 2)
        def _(p):
            for b in range(2):
                t = p * 2 + b

                @pl.when(t < nch)
                def _():
                    wait_gather(b)

                    @pl.when(t + 1 < nch)
                    def _():
                        wait_idx(t + 1, 1 - b)
                        start_gather(1 - b)

                    @pl.when(t + 2 < nch)
                    def _():
                        load_idx(t + 2, b)

                    @pl.when(t >= 2)
                    def _():
                        wait_write(t - 2, b)

                    for i in range(CHUNK):
                        for j in range(H // LANES):
                            sl = (b, i, pl.ds(j * LANES, LANES))
                            go[sl] = r1[sl] + r2[sl]

                    pltpu.async_copy(go.at[b],
                                     g_hbm.at[pl.ds(chunk_base(t), CHUNK)],
                                     osem)

        for t in (nch - 2, nch - 1):
            wait_write(t, t % 2)

    return pl.kernel(
        body,
        compiler_params=pltpu.CompilerParams(needs_layout_passes=False),
        out_type=jax.ShapeDtypeStruct((E, H), jnp.float32),
        mesh=mesh,
        scratch_types=[
            pltpu.VMEM((2, CHUNK), jnp.int32),
            pltpu.VMEM((2, CHUNK), jnp.int32),
            pltpu.VMEM((2, CHUNK, H), jnp.float32),
            pltpu.VMEM((2, CHUNK, H), jnp.float32),
            pltpu.VMEM((2, CHUNK, H), jnp.float32),
            pltpu.SemaphoreType.DMA,
            pltpu.SemaphoreType.DMA,
            pltpu.SemaphoreType.DMA,
        ],
    )


def _node_body(x_ref, msgp_ref, vol_ref, w0_ref, b0_ref, w1_ref, b1_ref,
               p_ref, q_ref, eb_ref, xout_ref, pa_ref, qb_ref):
    x = x_ref[...]
    m = msgp_ref[0] + msgp_ref[1]
    ni = jnp.concatenate([x, m], axis=1)
    f0 = jnp.dot(ni, w0_ref[...], preferred_element_type=jnp.float32) + b0_ref[...]
    f1 = jnp.dot(ni, w1_ref[...], preferred_element_type=jnp.float32) + b1_ref[...]
    vol = vol_ref[...]
    xo = jnp.where(vol == 0, f0, f1) + x
    xout_ref[...] = xo
    for c in range(3):
        pa_ref[c] = jnp.dot(xo, p_ref[c], preferred_element_type=jnp.float32) + eb_ref[c]
        qb_ref[c] = jnp.dot(xo, q_ref[c], preferred_element_type=jnp.float32)


@functools.lru_cache(maxsize=None)
def _make_stage2(N, H, BN=2000):
    grid = N // BN
    f32 = jnp.float32
    return pl.pallas_call(
        _node_body,
        grid=(grid,),
        in_specs=[
            pl.BlockSpec((BN, H), lambda i: (i, 0)),
            pl.BlockSpec((NC, BN, H), lambda i: (0, i, 0)),
            pl.BlockSpec((BN, 1), lambda i: (i, 0)),
            pl.BlockSpec((2 * H, H), lambda i: (0, 0)),
            pl.BlockSpec((1, H), lambda i: (0, 0)),
            pl.BlockSpec((2 * H, H), lambda i: (0, 0)),
            pl.BlockSpec((1, H), lambda i: (0, 0)),
            pl.BlockSpec((3, H, H), lambda i: (0, 0, 0)),
            pl.BlockSpec((3, H, H), lambda i: (0, 0, 0)),
            pl.BlockSpec((3, 1, H), lambda i: (0, 0, 0)),
        ],
        out_specs=[
            pl.BlockSpec((BN, H), lambda i: (i, 0)),
            pl.BlockSpec((3, BN, H), lambda i: (0, i, 0)),
            pl.BlockSpec((3, BN, H), lambda i: (0, i, 0)),
        ],
        out_shape=[
            jax.ShapeDtypeStruct((N, H), f32),
            jax.ShapeDtypeStruct((3, N, H), f32),
            jax.ShapeDtypeStruct((3, N, H), f32),
        ],
    )


def _edge_body(e_ref, g_ref, combo_ref, r_ref, out_ref):
    eb = e_ref[...]
    eh = eb.astype(jnp.bfloat16)
    r = r_ref[...].astype(jnp.bfloat16)
    m0 = jnp.dot(eh, r[0], preferred_element_type=jnp.float32)
    m1 = jnp.dot(eh, r[1], preferred_element_type=jnp.float32)
    m2 = jnp.dot(eh, r[2], preferred_element_type=jnp.float32)
    c = combo_ref[...]
    sel = jnp.where(c == 0, m0, jnp.where(c == 1, m1, m2))
    out_ref[...] = eb + jnp.where(c <= 2, g_ref[...] + sel, 0.0)


@functools.lru_cache(maxsize=None)
def _make_stage4(E, H, BE=4000):
    grid = E // BE
    return pl.pallas_call(
        _edge_body,
        grid=(grid,),
        in_specs=[
            pl.BlockSpec((BE, H), lambda i: (i, 0)),
            pl.BlockSpec((BE, H), lambda i: (i, 0)),
            pl.BlockSpec((BE, 1), lambda i: (i, 0)),
            pl.BlockSpec((3, H, H), lambda i: (0, 0, 0)),
        ],
        out_specs=pl.BlockSpec((BE, H), lambda i: (i, 0)),
        out_shape=jax.ShapeDtypeStruct((E, H), jnp.float32),
    )


def kernel(x, edge_index, e, volume_id, ne0_W, ne0_b, ne1_W, ne1_b,
           ee0_W, ee0_b, ee1_W, ee1_b, ee2_W, ee2_b):
    N, H = x.shape
    E = e.shape[0]
    start = edge_index[0]
    end = edge_index[1]

    msgp, combo, idx1, idx2 = _make_stage1(E, N, H)(e, start, end, volume_id)

    P = jnp.stack([ee0_W[:H], ee1_W[:H], ee2_W[:H]])
    Q = jnp.stack([ee0_W[H:2 * H], ee1_W[H:2 * H], ee2_W[H:2 * H]])
    R = jnp.stack([ee0_W[2 * H:], ee1_W[2 * H:], ee2_W[2 * H:]])
    ebias = jnp.stack([ee0_b, ee1_b, ee2_b])[:, None, :]

    x_out, PA, QB = _make_stage2(N, H)(
        x, msgp, volume_id[:, None], ne0_W, ne0_b[None], ne1_W, ne1_b[None],
        P, Q, ebias)

    Gv = _make_stage3(E, N, H)(
        PA.reshape(3 * N, H), QB.reshape(3 * N, H), idx1, idx2)

    e_out = _make_stage4(E, H)(e, Gv, combo[:, None], R)
    return (x_out, e_out)


# speedup vs baseline: 1.1881x; 1.1881x over previous
import functools

import jax
import jax.numpy as jnp
from jax import lax
from jax.experimental import pallas as pl
from jax.experimental.pallas import tpu as pltpu
from jax.experimental.pallas import tpu_sc as plsc

NC = 2
NS = 16
NW = NC * NS
LANES = 16
CHUNK = 80


@functools.lru_cache(maxsize=None)
def _make_stage1(E, N, H):
    EP = E // NW
    nch = EP // CHUNK
    RPT = (N // NS) & ~7
    tail = N - RPT * NS
    zrows = 48
    assert EP % CHUNK == 0 and RPT % zrows == 0 and tail % 8 == 0
    assert tail <= zrows

    mesh = plsc.VectorSubcoreMesh(core_axis_name="c", subcore_axis_name="s")

    def body(e_hbm, start_hbm, end_hbm, vol_hbm,
             msgp_hbm, combo_hbm, idx1_hbm, idx2_hbm,
             vol_v, erows, sidx, eidx, combo_v, i1_v, i2_v, zblk, msg_sh,
             lsem, ssem, wsem):
        cid = lax.axis_index("c")
        sid = lax.axis_index("s")
        wid = cid * NS + sid

        zero16 = jnp.zeros((LANES,), jnp.float32)
        for i in range(zrows):
            for j in range(H // LANES):
                zblk[i, pl.ds(j * LANES, LANES)] = zero16
        r0 = pl.multiple_of(sid * RPT, 8)
        for k in range(RPT // zrows):
            pltpu.sync_copy(zblk, msg_sh.at[pl.ds(r0 + k * zrows, zrows)])
        if tail:
            @pl.when(sid == 0)
            def _():
                pltpu.sync_copy(zblk.at[pl.ds(0, tail)],
                                msg_sh.at[pl.ds(RPT * NS, tail)])

        pltpu.sync_copy(vol_hbm, vol_v)
        plsc.subcore_barrier()

        def chunk_base(t):
            return pl.multiple_of(wid * EP + t * CHUNK, 8)

        def start_loads(t, b):
            base = chunk_base(t)
            pltpu.async_copy(start_hbm.at[pl.ds(base, CHUNK)], sidx.at[b], lsem)
            pltpu.async_copy(end_hbm.at[pl.ds(base, CHUNK)], eidx.at[b], lsem)
            pltpu.async_copy(e_hbm.at[pl.ds(base, CHUNK)], erows.at[b], lsem)

        def wait_loads(t, b):
            base = chunk_base(t)
            pltpu.make_async_copy(start_hbm.at[pl.ds(base, CHUNK)],
                                  sidx.at[b], lsem).wait()
            pltpu.make_async_copy(end_hbm.at[pl.ds(base, CHUNK)],
                                  eidx.at[b], lsem).wait()
            pltpu.make_async_copy(e_hbm.at[pl.ds(base, CHUNK)],
                                  erows.at[b], lsem).wait()

        def wait_scatter(b):
            pltpu.make_async_copy(erows.at[b], msg_sh.at[eidx.at[b]],
                                  ssem).wait()

        def wait_writes(t, b):
            base = chunk_base(t)
            pltpu.make_async_copy(combo_v.at[b],
                                  combo_hbm.at[pl.ds(base, CHUNK)], wsem).wait()
            pltpu.make_async_copy(i1_v.at[b],
                                  idx1_hbm.at[pl.ds(base, CHUNK)], wsem).wait()
            pltpu.make_async_copy(i2_v.at[b],
                                  idx2_hbm.at[pl.ds(base, CHUNK)], wsem).wait()

        start_loads(0, 0)

        @pl.loop(0, (nch + 1) // 2)
        def _(p):
            for b in range(2):
                t = p * 2 + b

                @pl.when(t < nch)
                def _():
                    base = chunk_base(t)
                    wait_loads(t, b)
                    pltpu.async_copy(erows.at[b], msg_sh.at[eidx.at[b]], ssem,
                                     add=True)

                    @pl.when(t >= 1)
                    def _():
                        wait_scatter(1 - b)

                    @pl.when(t + 1 < nch)
                    def _():
                        start_loads(t + 1, 1 - b)

                    @pl.when(t >= 2)
                    def _():
                        wait_writes(t - 2, b)

                    for j in range(CHUNK // LANES):
                        sl = (b, pl.ds(j * LANES, LANES))
                        s = sidx[sl]
                        d = eidx[sl]
                        vs = plsc.load_gather(vol_v, [s])
                        ve = plsc.load_gather(vol_v, [d])
                        code = vs * 2 + ve
                        c = jnp.where(code == 3, 2,
                                      jnp.where(code == 2, 3, code))
                        cc = jnp.where(c == 3, 0, c)
                        combo_v[sl] = c
                        i1_v[sl] = cc * N + s
                        i2_v[sl] = cc * N + d
                    pltpu.async_copy(combo_v.at[b],
                                     combo_hbm.at[pl.ds(base, CHUNK)], wsem)
                    pltpu.async_copy(i1_v.at[b],
                                     idx1_hbm.at[pl.ds(base, CHUNK)], wsem)
                    pltpu.async_copy(i2_v.at[b],
                                     idx2_hbm.at[pl.ds(base, CHUNK)], wsem)

        wait_scatter((nch - 1) % 2)
        for t in (nch - 2, nch - 1):
            wait_writes(t, t % 2)
        plsc.subcore_barrier()
        pltpu.sync_copy(msg_sh.at[pl.ds(r0, RPT)],
                        msgp_hbm.at[cid, pl.ds(r0, RPT)])
        if tail:
            @pl.when(sid == 0)
            def _():
                pltpu.sync_copy(msg_sh.at[pl.ds(RPT * NS, tail)],
                                msgp_hbm.at[cid, pl.ds(RPT * NS, tail)])

    return pl.kernel(
        body,
        compiler_params=pltpu.CompilerParams(needs_layout_passes=False),
        out_type=[
            jax.ShapeDtypeStruct((NC, N, H), jnp.float32),
            jax.ShapeDtypeStruct((E,), jnp.int32),
            jax.ShapeDtypeStruct((E,), jnp.int32),
            jax.ShapeDtypeStruct((E,), jnp.int32),
        ],
        mesh=mesh,
        scratch_types=[
            pltpu.VMEM((N,), jnp.int32),
            pltpu.VMEM((2, CHUNK, H), jnp.float32),
            pltpu.VMEM((2, CHUNK), jnp.int32),
            pltpu.VMEM((2, CHUNK), jnp.int32),
            pltpu.VMEM((2, CHUNK), jnp.int32),
            pltpu.VMEM((2, CHUNK), jnp.int32),
            pltpu.VMEM((2, CHUNK), jnp.int32),
            pltpu.VMEM((48, H), jnp.float32),
            pltpu.VMEM_SHARED((N, H), jnp.float32),
            pltpu.SemaphoreType.DMA,
            pltpu.SemaphoreType.DMA,
            pltpu.SemaphoreType.DMA,
        ],
    )


@functools.lru_cache(maxsize=None)
def _make_stage3(E, N, H):
    EP = E // NW
    nch = EP // CHUNK
    HW = H // 2

    mesh = plsc.VectorSubcoreMesh(core_axis_name="c", subcore_axis_name="s")

    def body(pa_hbm, qb_hbm, idx1_hbm, idx2_hbm, g_hbm,
             i1_v, i2_v, r1, g1sem, g2sem, isem, osem):
        cid = lax.axis_index("c")
        sid = lax.axis_index("s")
        wid = cid * NS + sid

        def chunk_base(t):
            return pl.multiple_of(wid * EP + t * CHUNK, 8)

        def load_idx(t, b):
            base = chunk_base(t)
            pltpu.async_copy(idx1_hbm.at[pl.ds(base, CHUNK)], i1_v.at[b], isem)
            pltpu.async_copy(idx2_hbm.at[pl.ds(base, CHUNK)], i2_v.at[b], isem)

        def wait_idx(t, b):
            base = chunk_base(t)
            pltpu.make_async_copy(idx1_hbm.at[pl.ds(base, CHUNK)],
                                  i1_v.at[b], isem).wait()
            pltpu.make_async_copy(idx2_hbm.at[pl.ds(base, CHUNK)],
                                  i2_v.at[b], isem).wait()

        def gather1(b):
            pltpu.async_copy(pa_hbm.at[i1_v.at[b]], r1.at[b], g1sem)

        def wait_gather1(b):
            pltpu.make_async_copy(pa_hbm.at[i1_v.at[b]], r1.at[b],
                                  g1sem).wait()

        def gather2(b):
            pltpu.async_copy(qb_hbm.at[i2_v.at[b]], r1.at[b], g2sem, add=True)

        def wait_gather2(b):
            pltpu.make_async_copy(qb_hbm.at[i2_v.at[b]], r1.at[b],
                                  g2sem).wait()

        def wait_write(t, b):
            pltpu.make_async_copy(r1.at[b],
                                  g_hbm.at[pl.ds(chunk_base(t), CHUNK)],
                                  osem).wait()

        load_idx(0, 0)
        wait_idx(0, 0)
        gather1(0)
        load_idx(1, 1)

        @pl.loop(0, (nch + 1) // 2)
        def _(p):
            for b in range(2):
                t = p * 2 + b

                @pl.when(t < nch)
                def _():
                    wait_gather1(b)
                    gather2(b)

                    @pl.when(t + 1 < nch)
                    def _():
                        wait_idx(t + 1, 1 - b)

                        @pl.when(t >= 1)
                        def _():
                            wait_write(t - 1, 1 - b)
                        gather1(1 - b)

                    wait_gather2(b)

                    @pl.when(t + 2 < nch)
                    def _():
                        load_idx(t + 2, b)

                    pltpu.async_copy(r1.at[b],
                                     g_hbm.at[pl.ds(chunk_base(t), CHUNK)],
                                     osem)

        for t in (nch - 2, nch - 1):
            wait_write(t, t % 2)

    return pl.kernel(
        body,
        compiler_params=pltpu.CompilerParams(needs_layout_passes=False),
        out_type=jax.ShapeDtypeStruct((E, H), jnp.float32),
        mesh=mesh,
        scratch_types=[
            pltpu.VMEM((2, CHUNK), jnp.int32),
            pltpu.VMEM((2, CHUNK), jnp.int32),
            pltpu.VMEM((2, CHUNK, H), jnp.float32),
            pltpu.SemaphoreType.DMA,
            pltpu.SemaphoreType.DMA,
            pltpu.SemaphoreType.DMA,
            pltpu.SemaphoreType.DMA,
        ],
    )


def _node_body(x_ref, msgp_ref, vol_ref, w0_ref, b0_ref, w1_ref, b1_ref,
               p_ref, q_ref, eb_ref, xout_ref, pa_ref, qb_ref):
    x = x_ref[...]
    m = msgp_ref[0] + msgp_ref[1]
    ni = jnp.concatenate([x, m], axis=1)
    f0 = jnp.dot(ni, w0_ref[...], preferred_element_type=jnp.float32) + b0_ref[...]
    f1 = jnp.dot(ni, w1_ref[...], preferred_element_type=jnp.float32) + b1_ref[...]
    vol = vol_ref[...]
    xo = jnp.where(vol == 0, f0, f1) + x
    xout_ref[...] = xo
    for c in range(3):
        pa_ref[c] = jnp.dot(xo, p_ref[c], preferred_element_type=jnp.float32) + eb_ref[c]
        qb_ref[c] = jnp.dot(xo, q_ref[c], preferred_element_type=jnp.float32)


@functools.lru_cache(maxsize=None)
def _make_stage2(N, H, BN=2000):
    grid = N // BN
    f32 = jnp.float32
    return pl.pallas_call(
        _node_body,
        grid=(grid,),
        in_specs=[
            pl.BlockSpec((BN, H), lambda i: (i, 0)),
            pl.BlockSpec((NC, BN, H), lambda i: (0, i, 0)),
            pl.BlockSpec((BN, 1), lambda i: (i, 0)),
            pl.BlockSpec((2 * H, H), lambda i: (0, 0)),
            pl.BlockSpec((1, H), lambda i: (0, 0)),
            pl.BlockSpec((2 * H, H), lambda i: (0, 0)),
            pl.BlockSpec((1, H), lambda i: (0, 0)),
            pl.BlockSpec((3, H, H), lambda i: (0, 0, 0)),
            pl.BlockSpec((3, H, H), lambda i: (0, 0, 0)),
            pl.BlockSpec((3, 1, H), lambda i: (0, 0, 0)),
        ],
        out_specs=[
            pl.BlockSpec((BN, H), lambda i: (i, 0)),
            pl.BlockSpec((3, BN, H), lambda i: (0, i, 0)),
            pl.BlockSpec((3, BN, H), lambda i: (0, i, 0)),
        ],
        out_shape=[
            jax.ShapeDtypeStruct((N, H), f32),
            jax.ShapeDtypeStruct((3, N, H), f32),
            jax.ShapeDtypeStruct((3, N, H), f32),
        ],
    )


def _edge_body(e_ref, g_ref, combo_ref, r_ref, out_ref):
    eb = e_ref[...]
    eh = eb.astype(jnp.bfloat16)
    r = r_ref[...].astype(jnp.bfloat16)
    m0 = jnp.dot(eh, r[0], preferred_element_type=jnp.float32)
    m1 = jnp.dot(eh, r[1], preferred_element_type=jnp.float32)
    m2 = jnp.dot(eh, r[2], preferred_element_type=jnp.float32)
    c = combo_ref[...]
    sel = jnp.where(c == 0, m0, jnp.where(c == 1, m1, m2))
    out_ref[...] = eb + jnp.where(c <= 2, g_ref[...] + sel, 0.0)


@functools.lru_cache(maxsize=None)
def _make_stage4(E, H, BE=4000):
    grid = E // BE
    return pl.pallas_call(
        _edge_body,
        grid=(grid,),
        in_specs=[
            pl.BlockSpec((BE, H), lambda i: (i, 0)),
            pl.BlockSpec((BE, H), lambda i: (i, 0)),
            pl.BlockSpec((BE, 1), lambda i: (i, 0)),
            pl.BlockSpec((3, H, H), lambda i: (0, 0, 0)),
        ],
        out_specs=pl.BlockSpec((BE, H), lambda i: (i, 0)),
        out_shape=jax.ShapeDtypeStruct((E, H), jnp.float32),
    )


def kernel(x, edge_index, e, volume_id, ne0_W, ne0_b, ne1_W, ne1_b,
           ee0_W, ee0_b, ee1_W, ee1_b, ee2_W, ee2_b):
    N, H = x.shape
    E = e.shape[0]
    start = edge_index[0]
    end = edge_index[1]

    msgp, combo, idx1, idx2 = _make_stage1(E, N, H)(e, start, end, volume_id)

    P = jnp.stack([ee0_W[:H], ee1_W[:H], ee2_W[:H]])
    Q = jnp.stack([ee0_W[H:2 * H], ee1_W[H:2 * H], ee2_W[H:2 * H]])
    R = jnp.stack([ee0_W[2 * H:], ee1_W[2 * H:], ee2_W[2 * H:]])
    ebias = jnp.stack([ee0_b, ee1_b, ee2_b])[:, None, :]

    x_out, PA, QB = _make_stage2(N, H)(
        x, msgp, volume_id[:, None], ne0_W, ne0_b[None], ne1_W, ne1_b[None],
        P, Q, ebias)

    Gv = _make_stage3(E, N, H)(
        PA.reshape(3 * N, H), QB.reshape(3 * N, H), idx1, idx2)

    e_out = _make_stage4(E, H)(e, Gv, combo[:, None], R)
    return (x_out, e_out)
